# baseline (speedup 1.0000x reference)
import jax
import jax.numpy as jnp
from jax import lax
from jax.experimental import pallas as pl
from jax.experimental.pallas import tpu as pltpu
from jax.experimental.pallas import tpu_sc as plsc

N = 10000
D = 128
E = 320000
NC = 2
NS = 16
NW = NC * NS
EPT = E // NW
CH = 125
NCHUNK = EPT // CH
GRP = 8
NGRP = NCHUNK // GRP
NP = 10240
RPT = NP // NS
DW = 16

_T15_ROWS = (D, D, D, 16, 0)

_mesh = plsc.VectorSubcoreMesh(core_axis_name="c", subcore_axis_name="s",
                               num_cores=NC, num_subcores=NS)


def _fill_rows(ref, nrows, width, val):
    def body(i, carry):
        for p in range(width // 16):
            ref[i, pl.ds(p * 16, 16)] = jnp.full((16,), val, jnp.float32)
        return carry

    lax.fori_loop(0, nrows, body, None)


def _each_own_chunk(s, fn):
    for j in range(RPT // D):
        off = pl.multiple_of(s * RPT + j * D, D)
        t15 = _T15_ROWS[j]
        if t15 == D:
            fn(j, off, D)
        else:
            @pl.when(s < NS - 1)
            def _(j=j, off=off):
                fn(j, off, D)

            if t15 > 0:
                @pl.when(s == NS - 1)
                def _(j=j, off=off, t15=t15):
                    fn(j, off, t15)


def _deg_body(dst_hbm, out_hbm, idx_v, zb_v, acc_sh, sem):
    c = lax.axis_index("c")
    s = lax.axis_index("s")
    w = s * NC + c
    pltpu.sync_copy(dst_hbm.at[w], idx_v)
    _fill_rows(zb_v, D, DW, 0.0)

    for j in range(RPT // D):
        off = pl.multiple_of(s * RPT + j * D, D)
        pltpu.sync_copy(zb_v, acc_sh.at[pl.ds(off, D)])
    _fill_rows(zb_v, D, DW, 1.0)
    plsc.subcore_barrier()
    ones_rows = zb_v.at[pl.ds(0, CH)]

    def step(i, carry):
        pltpu.sync_copy(ones_rows, acc_sh.at[idx_v.at[i]], add=True)
        return carry

    lax.fori_loop(0, NCHUNK, step, None)
    plsc.subcore_barrier()

    for j in range(RPT // D):
        off = pl.multiple_of(s * RPT + j * D, D)
        pltpu.sync_copy(acc_sh.at[pl.ds(off, D)], zb_v)
        pltpu.sync_copy(zb_v, out_hbm.at[c, pl.ds(s * RPT + j * D, D)])


_deg_call = pl.kernel(
    _deg_body,
    out_type=jax.ShapeDtypeStruct((NC, NP, DW), jnp.float32),
    mesh=_mesh,
    scratch_types=[
        pltpu.VMEM((NCHUNK, CH), jnp.int32),
        pltpu.VMEM((D, DW), jnp.float32),
        pltpu.VMEM_SHARED((NP, DW), jnp.float32),
        pltpu.SemaphoreType.DMA,
    ],
)


def _agg_body(hs_hbm, src_hbm, dst_hbm, out_hbm,
              idx_s, idx_d, rows_a, rows_b, acc_sh, sem_a, sem_b):
    c = lax.axis_index("c")
    s = lax.axis_index("s")
    w = s * NC + c
    pltpu.sync_copy(src_hbm.at[w], idx_s)
    pltpu.sync_copy(dst_hbm.at[w, pl.ds(0, GRP)], idx_d)
    _fill_rows(rows_a, D, D, 0.0)

    for j in range(RPT // D):
        off = pl.multiple_of(s * RPT + j * D, D)
        pltpu.sync_copy(rows_a, acc_sh.at[pl.ds(off, D)])
    plsc.subcore_barrier()
    bufs = (rows_a.at[pl.ds(0, CH)], rows_b.at[pl.ds(0, CH)])
    sems = (sem_a, sem_b)
    pltpu.async_copy(hs_hbm.at[idx_s.at[0]], bufs[0], sems[0])

    def group(g, carry):
        base = g * GRP
        for r in range(GRP):
            cur = r % 2
            nxt = 1 - cur
            nxt_chunk = base + r + 1

            @pl.when(nxt_chunk < NCHUNK)
            def _():
                pltpu.async_copy(hs_hbm.at[idx_s.at[nxt_chunk]],
                                 bufs[nxt], sems[nxt])

            pltpu.make_async_copy(hs_hbm.at[idx_s.at[0]], bufs[cur],
                                  sems[cur]).wait()
            pltpu.sync_copy(bufs[cur], acc_sh.at[idx_d.at[r]], add=True)

        @pl.when(g + 1 < NGRP)
        def _():
            off = pl.multiple_of((g + 1) * GRP, GRP)
            pltpu.sync_copy(dst_hbm.at[w, pl.ds(off, GRP)], idx_d)

        return carry

    lax.fori_loop(0, NGRP, group, None)
    plsc.subcore_barrier()

    for j in range(RPT // D):
        off = pl.multiple_of(s * RPT + j * D, D)
        pltpu.sync_copy(acc_sh.at[pl.ds(off, D)], rows_a)
        pltpu.sync_copy(rows_a, out_hbm.at[c, pl.ds(s * RPT + j * D, D)])


_agg_call = pl.kernel(
    _agg_body,
    out_type=jax.ShapeDtypeStruct((NC, NP, D), jnp.float32),
    mesh=_mesh,
    scratch_types=[
        pltpu.VMEM((NCHUNK, CH), jnp.int32),
        pltpu.VMEM((GRP, CH), jnp.int32),
        pltpu.VMEM((D, D), jnp.float32),
        pltpu.VMEM((D, D), jnp.float32),
        pltpu.VMEM_SHARED((NP, D), jnp.float32),
        pltpu.SemaphoreType.DMA,
        pltpu.SemaphoreType.DMA,
    ],
)

BN = 80


def _dis_of(dacc_ref):
    deg = dacc_ref[0] + dacc_ref[1] + 1.0
    return lax.rsqrt(deg)[:, 0:1]


def _t1_body(x_ref, w_ref, dacc_ref, o_ref):
    ds = _dis_of(dacc_ref)
    h = jnp.dot(x_ref[...], w_ref[...], preferred_element_type=jnp.float32)
    o_ref[...] = h * ds


def _t2_body(agg_ref, h1s_ref, dacc_ref, b_ref, w_ref, o_ref):
    ds = _dis_of(dacc_ref)
    z = jnp.maximum(ds * (agg_ref[0] + agg_ref[1] + h1s_ref[...]) + b_ref[...],
                    0.0)
    o_ref[...] = jnp.dot(z, w_ref[...], preferred_element_type=jnp.float32) * ds


def _t3_body(agg_ref, h2s_ref, dacc_ref, b_ref, o_ref):
    ds = _dis_of(dacc_ref)
    o_ref[...] = ds * (agg_ref[0] + agg_ref[1] + h2s_ref[...]) + b_ref[...]


_row_spec = pl.BlockSpec((BN, D), lambda i: (i, 0))
_agg_spec = pl.BlockSpec((NC, BN, D), lambda i: (0, i, 0))
_dacc_spec = pl.BlockSpec((NC, BN, DW), lambda i: (0, i, 0))
_w_spec = pl.BlockSpec((D, D), lambda i: (0, 0))
_b_spec = pl.BlockSpec((D,), lambda i: (0,))
_out_struct = jax.ShapeDtypeStruct((N, D), jnp.float32)

_t1_call = pl.pallas_call(
    _t1_body, grid=(N // BN,),
    in_specs=[_row_spec, _w_spec, _dacc_spec],
    out_specs=_row_spec, out_shape=_out_struct)

_t2_call = pl.pallas_call(
    _t2_body, grid=(N // BN,),
    in_specs=[_agg_spec, _row_spec, _dacc_spec, _b_spec, _w_spec],
    out_specs=_row_spec, out_shape=_out_struct)

_t3_call = pl.pallas_call(
    _t3_body, grid=(N // BN,),
    in_specs=[_agg_spec, _row_spec, _dacc_spec, _b_spec],
    out_specs=_row_spec, out_shape=_out_struct)


@jax.jit
def kernel(x, edge_index, W1, b1, W2, b2):
    src = edge_index[0].reshape(NW, NCHUNK, CH)
    dst = edge_index[1].reshape(NW, NCHUNK, CH)
    dacc = _deg_call(dst)
    h1s = _t1_call(x, W1, dacc)
    agg1 = _agg_call(h1s, src, dst)
    h2s = _t2_call(agg1, h1s, dacc, b1, W2)
    agg2 = _agg_call(h2s, src, dst)
    return _t3_call(agg2, h2s, dacc, b2)


# speedup vs baseline: 1.4771x; 1.4771x over previous
import jax
import jax.numpy as jnp
from jax import lax
from jax.experimental import pallas as pl
from jax.experimental.pallas import tpu as pltpu
from jax.experimental.pallas import tpu_sc as plsc

N = 10000
D = 128
E = 320000
NC = 2
NS = 16
NW = NC * NS
EPT = E // NW
CH = 125
NCHUNK = EPT // CH
GRP = 8
NGRP = NCHUNK // GRP
NP = 10240
RPT = NP // NS
DW = 16

_T15_ROWS = (D, D, D, 16, 0)

_mesh = plsc.VectorSubcoreMesh(core_axis_name="c", subcore_axis_name="s",
                               num_cores=NC, num_subcores=NS)


def _fill_rows(ref, nrows, width, val):
    def body(i, carry):
        for p in range(width // 16):
            ref[i, pl.ds(p * 16, 16)] = jnp.full((16,), val, jnp.float32)
        return carry

    lax.fori_loop(0, nrows, body, None)


def _each_own_chunk(s, fn):
    for j in range(RPT // D):
        off = pl.multiple_of(s * RPT + j * D, D)
        t15 = _T15_ROWS[j]
        if t15 == D:
            fn(j, off, D)
        else:
            @pl.when(s < NS - 1)
            def _(j=j, off=off):
                fn(j, off, D)

            if t15 > 0:
                @pl.when(s == NS - 1)
                def _(j=j, off=off, t15=t15):
                    fn(j, off, t15)


def _deg_body(dst_hbm, out_hbm, idx_v, zb_v, acc_sh, sem):
    c = lax.axis_index("c")
    s = lax.axis_index("s")
    w = s * NC + c
    pltpu.sync_copy(dst_hbm.at[w], idx_v)
    _fill_rows(zb_v, D, DW, 0.0)

    for j in range(RPT // D):
        off = pl.multiple_of(s * RPT + j * D, D)
        pltpu.sync_copy(zb_v, acc_sh.at[pl.ds(off, D)])
    _fill_rows(zb_v, D, DW, 1.0)
    plsc.subcore_barrier()
    ones_rows = zb_v.at[pl.ds(0, CH)]

    def step(i, carry):
        pltpu.sync_copy(ones_rows, acc_sh.at[idx_v.at[i]], add=True)
        return carry

    lax.fori_loop(0, NCHUNK, step, None)
    plsc.subcore_barrier()

    for j in range(RPT // D):
        off = pl.multiple_of(s * RPT + j * D, D)
        pltpu.sync_copy(acc_sh.at[pl.ds(off, D)], zb_v)
        pltpu.sync_copy(zb_v, out_hbm.at[c, pl.ds(s * RPT + j * D, D)])


_deg_call = pl.kernel(
    _deg_body,
    out_type=jax.ShapeDtypeStruct((NC, NP, DW), jnp.float32),
    mesh=_mesh,
    scratch_types=[
        pltpu.VMEM((NCHUNK, CH), jnp.int32),
        pltpu.VMEM((D, DW), jnp.float32),
        pltpu.VMEM_SHARED((NP, DW), jnp.float32),
        pltpu.SemaphoreType.DMA,
    ],
)


def _agg_body(hs_hbm, src_hbm, dst_hbm, out_hbm,
              idx_s, idx_d, rows_a, rows_b, acc_sh, sem_a, sem_b):
    c = lax.axis_index("c")
    s = lax.axis_index("s")
    w = s * NC + c
    pltpu.sync_copy(src_hbm.at[w], idx_s)
    pltpu.sync_copy(dst_hbm.at[w, pl.ds(0, GRP)], idx_d)
    _fill_rows(rows_a, D, D, 0.0)

    for j in range(RPT // D):
        off = pl.multiple_of(s * RPT + j * D, D)
        pltpu.sync_copy(rows_a, acc_sh.at[pl.ds(off, D)])
    plsc.subcore_barrier()
    bufs = (rows_a.at[pl.ds(0, CH)], rows_b.at[pl.ds(0, CH)])
    sems = (sem_a, sem_b)
    pltpu.async_copy(hs_hbm.at[idx_s.at[0]], bufs[0], sems[0])

    def group(g, carry):
        base = g * GRP
        for r in range(GRP):
            cur = r % 2
            nxt = 1 - cur
            nxt_chunk = base + r + 1

            @pl.when(nxt_chunk < NCHUNK)
            def _():
                pltpu.async_copy(hs_hbm.at[idx_s.at[nxt_chunk]],
                                 bufs[nxt], sems[nxt])

            pltpu.make_async_copy(hs_hbm.at[idx_s.at[0]], bufs[cur],
                                  sems[cur]).wait()
            pltpu.sync_copy(bufs[cur], acc_sh.at[idx_d.at[r]], add=True)

        @pl.when(g + 1 < NGRP)
        def _():
            off = pl.multiple_of((g + 1) * GRP, GRP)
            pltpu.sync_copy(dst_hbm.at[w, pl.ds(off, GRP)], idx_d)

        return carry

    lax.fori_loop(0, NGRP, group, None)
    plsc.subcore_barrier()

    for j in range(RPT // D):
        off = pl.multiple_of(s * RPT + j * D, D)
        pltpu.sync_copy(acc_sh.at[pl.ds(off, D)], rows_a)
        pltpu.sync_copy(rows_a, out_hbm.at[c, pl.ds(s * RPT + j * D, D)])


_agg_call = pl.kernel(
    _agg_body,
    out_type=jax.ShapeDtypeStruct((NC, NP, D), jnp.float32),
    mesh=_mesh,
    scratch_types=[
        pltpu.VMEM((NCHUNK, CH), jnp.int32),
        pltpu.VMEM((GRP, CH), jnp.int32),
        pltpu.VMEM((D, D), jnp.float32),
        pltpu.VMEM((D, D), jnp.float32),
        pltpu.VMEM_SHARED((NP, D), jnp.float32),
        pltpu.SemaphoreType.DMA,
        pltpu.SemaphoreType.DMA,
    ],
)

BN = 1000


def _dis_of(dacc_ref):
    deg = dacc_ref[0] + dacc_ref[1] + 1.0
    return lax.rsqrt(deg)[:, 0:1]


def _t1_body(x_ref, w_ref, dacc_ref, o_ref):
    ds = _dis_of(dacc_ref)
    h = jnp.dot(x_ref[...], w_ref[...], preferred_element_type=jnp.float32)
    o_ref[...] = h * ds


def _t2_body(agg_ref, h1s_ref, dacc_ref, b_ref, w_ref, o_ref):
    ds = _dis_of(dacc_ref)
    z = jnp.maximum(ds * (agg_ref[0] + agg_ref[1] + h1s_ref[...]) + b_ref[...],
                    0.0)
    o_ref[...] = jnp.dot(z, w_ref[...], preferred_element_type=jnp.float32) * ds


def _t3_body(agg_ref, h2s_ref, dacc_ref, b_ref, o_ref):
    ds = _dis_of(dacc_ref)
    o_ref[...] = ds * (agg_ref[0] + agg_ref[1] + h2s_ref[...]) + b_ref[...]


_row_spec = pl.BlockSpec((BN, D), lambda i: (i, 0))
_agg_spec = pl.BlockSpec((NC, BN, D), lambda i: (0, i, 0))
_dacc_spec = pl.BlockSpec((NC, BN, DW), lambda i: (0, i, 0))
_w_spec = pl.BlockSpec((D, D), lambda i: (0, 0))
_b_spec = pl.BlockSpec((D,), lambda i: (0,))
_out_struct = jax.ShapeDtypeStruct((N, D), jnp.float32)

_t1_call = pl.pallas_call(
    _t1_body, grid=(N // BN,),
    in_specs=[_row_spec, _w_spec, _dacc_spec],
    out_specs=_row_spec, out_shape=_out_struct)

_t2_call = pl.pallas_call(
    _t2_body, grid=(N // BN,),
    in_specs=[_agg_spec, _row_spec, _dacc_spec, _b_spec, _w_spec],
    out_specs=_row_spec, out_shape=_out_struct)

_t3_call = pl.pallas_call(
    _t3_body, grid=(N // BN,),
    in_specs=[_agg_spec, _row_spec, _dacc_spec, _b_spec],
    out_specs=_row_spec, out_shape=_out_struct)


@jax.jit
def kernel(x, edge_index, W1, b1, W2, b2):
    src = edge_index[0].reshape(NW, NCHUNK, CH)
    dst = edge_index[1].reshape(NW, NCHUNK, CH)
    dacc = _deg_call(dst)[:, :N]
    h1s = _t1_call(x, W1, dacc)
    agg1 = _agg_call(h1s, src, dst)
    h2s = _t2_call(agg1[:, :N], h1s, dacc, b1, W2)
    agg2 = _agg_call(h2s, src, dst)
    return _t3_call(agg2[:, :N], h2s, dacc, b2)


# speedup vs baseline: 1.4951x; 1.0122x over previous
import jax
import jax.numpy as jnp
from jax import lax
from jax.experimental import pallas as pl
from jax.experimental.pallas import tpu as pltpu
from jax.experimental.pallas import tpu_sc as plsc

N = 10000
D = 128
E = 320000
NC = 2
NS = 16
NW = NC * NS
EPT = E // NW
CH = 125
NCHUNK = EPT // CH
GRP = 8
NGRP = NCHUNK // GRP
NP = 10240
RPT = NP // NS
DW = 16

_T15_ROWS = (D, D, D, 16, 0)

_mesh = plsc.VectorSubcoreMesh(core_axis_name="c", subcore_axis_name="s",
                               num_cores=NC, num_subcores=NS)


def _fill_rows(ref, nrows, width, val):
    def body(i, carry):
        for p in range(width // 16):
            ref[i, pl.ds(p * 16, 16)] = jnp.full((16,), val, jnp.float32)
        return carry

    lax.fori_loop(0, nrows, body, None)


def _each_own_chunk(s, fn):
    for j in range(RPT // D):
        off = pl.multiple_of(s * RPT + j * D, D)
        t15 = _T15_ROWS[j]
        if t15 == D:
            fn(j, off, D)
        else:
            @pl.when(s < NS - 1)
            def _(j=j, off=off):
                fn(j, off, D)

            if t15 > 0:
                @pl.when(s == NS - 1)
                def _(j=j, off=off, t15=t15):
                    fn(j, off, t15)


def _deg_body(dst_hbm, out_hbm, idx_v, zb_v, acc_sh, sem):
    c = lax.axis_index("c")
    s = lax.axis_index("s")
    w = s * NC + c
    pltpu.sync_copy(dst_hbm.at[w], idx_v)
    _fill_rows(zb_v, D, DW, 0.0)

    for j in range(RPT // D):
        off = pl.multiple_of(s * RPT + j * D, D)
        pltpu.sync_copy(zb_v, acc_sh.at[pl.ds(off, D)])
    _fill_rows(zb_v, D, DW, 1.0)
    plsc.subcore_barrier()
    ones_rows = zb_v.at[pl.ds(0, CH)]

    def step(g, carry):
        for r in range(GRP):
            pltpu.async_copy(ones_rows, acc_sh.at[idx_v.at[g * GRP + r]],
                             sem, add=True)
        for r in range(GRP):
            pltpu.make_async_copy(ones_rows, acc_sh.at[idx_v.at[0]],
                                  sem).wait()
        return carry

    lax.fori_loop(0, NGRP, step, None)
    plsc.subcore_barrier()

    for j in range(RPT // D):
        off = pl.multiple_of(s * RPT + j * D, D)
        pltpu.sync_copy(acc_sh.at[pl.ds(off, D)], zb_v)
        pltpu.sync_copy(zb_v, out_hbm.at[c, pl.ds(s * RPT + j * D, D)])


_deg_call = pl.kernel(
    _deg_body,
    out_type=jax.ShapeDtypeStruct((NC, NP, DW), jnp.float32),
    mesh=_mesh,
    scratch_types=[
        pltpu.VMEM((NCHUNK, CH), jnp.int32),
        pltpu.VMEM((D, DW), jnp.float32),
        pltpu.VMEM_SHARED((NP, DW), jnp.float32),
        pltpu.SemaphoreType.DMA,
    ],
)


def _agg_body(hs_hbm, src_hbm, dst_hbm, out_hbm,
              idx_s, idx_d, rows_a, rows_b, acc_sh, sem_a, sem_b):
    c = lax.axis_index("c")
    s = lax.axis_index("s")
    w = s * NC + c
    pltpu.sync_copy(src_hbm.at[w], idx_s)
    pltpu.sync_copy(dst_hbm.at[w, pl.ds(0, GRP)], idx_d)
    _fill_rows(rows_a, D, D, 0.0)

    for j in range(RPT // D):
        off = pl.multiple_of(s * RPT + j * D, D)
        pltpu.sync_copy(rows_a, acc_sh.at[pl.ds(off, D)])
    plsc.subcore_barrier()
    bufs = (rows_a.at[pl.ds(0, CH)], rows_b.at[pl.ds(0, CH)])
    sems = (sem_a, sem_b)
    pltpu.async_copy(hs_hbm.at[idx_s.at[0]], bufs[0], sems[0])

    def group(g, carry):
        base = g * GRP
        for r in range(GRP):
            cur = r % 2
            nxt = 1 - cur
            nxt_chunk = base + r + 1

            @pl.when(nxt_chunk < NCHUNK)
            def _():
                pltpu.async_copy(hs_hbm.at[idx_s.at[nxt_chunk]],
                                 bufs[nxt], sems[nxt])

            pltpu.make_async_copy(hs_hbm.at[idx_s.at[0]], bufs[cur],
                                  sems[cur]).wait()
            pltpu.sync_copy(bufs[cur], acc_sh.at[idx_d.at[r]], add=True)

        @pl.when(g + 1 < NGRP)
        def _():
            off = pl.multiple_of((g + 1) * GRP, GRP)
            pltpu.sync_copy(dst_hbm.at[w, pl.ds(off, GRP)], idx_d)

        return carry

    lax.fori_loop(0, NGRP, group, None)
    plsc.subcore_barrier()

    for j in range(RPT // D):
        off = pl.multiple_of(s * RPT + j * D, D)
        pltpu.sync_copy(acc_sh.at[pl.ds(off, D)], rows_a)
        pltpu.sync_copy(rows_a, out_hbm.at[c, pl.ds(s * RPT + j * D, D)])


_agg_call = pl.kernel(
    _agg_body,
    out_type=jax.ShapeDtypeStruct((NC, NP, D), jnp.float32),
    mesh=_mesh,
    scratch_types=[
        pltpu.VMEM((NCHUNK, CH), jnp.int32),
        pltpu.VMEM((GRP, CH), jnp.int32),
        pltpu.VMEM((D, D), jnp.float32),
        pltpu.VMEM((D, D), jnp.float32),
        pltpu.VMEM_SHARED((NP, D), jnp.float32),
        pltpu.SemaphoreType.DMA,
        pltpu.SemaphoreType.DMA,
    ],
)

BN = 1000


def _dis_of(dacc_ref):
    deg = dacc_ref[0] + dacc_ref[1] + 1.0
    return lax.rsqrt(deg)[:, 0:1]


def _t1_body(x_ref, w_ref, dacc_ref, o_ref):
    ds = _dis_of(dacc_ref)
    h = jnp.dot(x_ref[...], w_ref[...], preferred_element_type=jnp.float32)
    o_ref[...] = h * ds


def _t2_body(agg_ref, h1s_ref, dacc_ref, b_ref, w_ref, o_ref):
    ds = _dis_of(dacc_ref)
    z = jnp.maximum(ds * (agg_ref[0] + agg_ref[1] + h1s_ref[...]) + b_ref[...],
                    0.0)
    o_ref[...] = jnp.dot(z, w_ref[...], preferred_element_type=jnp.float32) * ds


def _t3_body(agg_ref, h2s_ref, dacc_ref, b_ref, o_ref):
    ds = _dis_of(dacc_ref)
    o_ref[...] = ds * (agg_ref[0] + agg_ref[1] + h2s_ref[...]) + b_ref[...]


_row_spec = pl.BlockSpec((BN, D), lambda i: (i, 0))
_agg_spec = pl.BlockSpec((NC, BN, D), lambda i: (0, i, 0))
_dacc_spec = pl.BlockSpec((NC, BN, DW), lambda i: (0, i, 0))
_w_spec = pl.BlockSpec((D, D), lambda i: (0, 0))
_b_spec = pl.BlockSpec((D,), lambda i: (0,))
_out_struct = jax.ShapeDtypeStruct((N, D), jnp.float32)

_t1_call = pl.pallas_call(
    _t1_body, grid=(N // BN,),
    in_specs=[_row_spec, _w_spec, _dacc_spec],
    out_specs=_row_spec, out_shape=_out_struct)

_t2_call = pl.pallas_call(
    _t2_body, grid=(N // BN,),
    in_specs=[_agg_spec, _row_spec, _dacc_spec, _b_spec, _w_spec],
    out_specs=_row_spec, out_shape=_out_struct)

_t3_call = pl.pallas_call(
    _t3_body, grid=(N // BN,),
    in_specs=[_agg_spec, _row_spec, _dacc_spec, _b_spec],
    out_specs=_row_spec, out_shape=_out_struct)


@jax.jit
def kernel(x, edge_index, W1, b1, W2, b2):
    src = edge_index[0].reshape(NW, NCHUNK, CH)
    dst = edge_index[1].reshape(NW, NCHUNK, CH)
    dacc = _deg_call(dst)[:, :N]
    h1s = _t1_call(x, W1, dacc)
    agg1 = _agg_call(h1s, src, dst)
    h2s = _t2_call(agg1[:, :N], h1s, dacc, b1, W2)
    agg2 = _agg_call(h2s, src, dst)
    return _t3_call(agg2[:, :N], h2s, dacc, b2)
